# baseline (speedup 1.0000x reference)
import functools
import math

import jax
import jax.numpy as jnp
from jax import lax
from jax.experimental import pallas as pl
from jax.experimental.pallas import tpu as pltpu
from jax.experimental.pallas import tpu_sc as plsc

D_MODEL = 64
SCALE = math.sqrt(D_MODEL)
LANES = 16
NUM_CORES = 2
NUM_SUBCORES = 16
NUM_WORKERS = NUM_CORES * NUM_SUBCORES

B_TOTAL = 4096 * 200
ROWS_PER_WORKER = B_TOTAL // NUM_WORKERS
CHUNK = 512
NCHUNKS = ROWS_PER_WORKER // CHUNK


def _emb_kernel(lut_hbm, idx_hbm, out_hbm, idx_v, rows_v, sem):
    wid = lax.axis_index("s") * NUM_CORES + lax.axis_index("c")
    base = wid * ROWS_PER_WORKER

    @pl.loop(0, NCHUNKS)
    def _(k):
        off = base + k * CHUNK
        pltpu.sync_copy(idx_hbm.at[pl.ds(off, CHUNK)], idx_v)
        pltpu.async_copy(lut_hbm.at[idx_v], rows_v, sem).wait()

        @pl.loop(0, CHUNK)
        def _(r):
            for c in range(D_MODEL // LANES):
                slc = pl.ds(c * LANES, LANES)
                rows_v[r, slc] = rows_v[r, slc] * SCALE

        pltpu.sync_copy(rows_v, out_hbm.at[pl.ds(off, CHUNK)])


@jax.jit
def kernel(x, lut):
    idx = x.reshape(B_TOTAL)
    mesh = plsc.VectorSubcoreMesh(core_axis_name="c", subcore_axis_name="s")
    run = pl.kernel(
        _emb_kernel,
        out_type=jax.ShapeDtypeStruct((B_TOTAL, D_MODEL), jnp.float32),
        mesh=mesh,
        scratch_types=[
            pltpu.VMEM((CHUNK,), jnp.int32),
            pltpu.VMEM((CHUNK, D_MODEL), jnp.float32),
            pltpu.SemaphoreType.DMA,
        ],
        compiler_params=pltpu.CompilerParams(use_tc_tiling_on_sc=False),
    )
    out = run(lut, idx)
    return out.reshape(x.shape[0], x.shape[1], D_MODEL)


# speedup vs baseline: 1.1359x; 1.1359x over previous
import functools
import math

import jax
import jax.numpy as jnp
from jax import lax
from jax.experimental import pallas as pl
from jax.experimental.pallas import tpu as pltpu
from jax.experimental.pallas import tpu_sc as plsc

D_MODEL = 64
SCALE = math.sqrt(D_MODEL)
LANES = 16
NUM_CORES = 2
NUM_SUBCORES = 16
NUM_WORKERS = NUM_CORES * NUM_SUBCORES

B_TOTAL = 4096 * 200
ROWS_PER_WORKER = B_TOTAL // NUM_WORKERS
CHUNK = 256
NCHUNKS = ROWS_PER_WORKER // CHUNK
NBUF = 2


def _emb_kernel(lut_hbm, idx_hbm, out_hbm,
                idx_v, g0, g1, o0, o1, gs0, gs1, os0, os1):
    wid = lax.axis_index("s") * NUM_CORES + lax.axis_index("c")
    base = wid * ROWS_PER_WORKER

    gbufs, obufs = (g0, g1), (o0, o1)
    gsems, osems = (gs0, gs1), (os0, os1)

    pltpu.sync_copy(idx_hbm.at[pl.ds(base, ROWS_PER_WORKER)], idx_v)

    def start_gather(k, b):
        pltpu.make_async_copy(
            lut_hbm.at[idx_v.at[pl.ds(k * CHUNK, CHUNK)]], gbufs[b], gsems[b]
        ).start()

    def wait_gather(b):
        pltpu.make_async_copy(
            lut_hbm.at[idx_v.at[pl.ds(0, CHUNK)]], gbufs[b], gsems[b]
        ).wait()

    def start_out(k, b):
        pltpu.make_async_copy(
            obufs[b], out_hbm.at[pl.ds(base + k * CHUNK, CHUNK)], osems[b]
        ).start()

    def wait_out(b):
        pltpu.make_async_copy(
            obufs[b], out_hbm.at[pl.ds(base, CHUNK)], osems[b]
        ).wait()

    for b in range(NBUF):
        start_gather(b, b)

    @pl.loop(0, NCHUNKS, step=NBUF)
    def _(k0):
        for b in range(NBUF):
            k = k0 + b
            wait_gather(b)

            @pl.when(k0 > 0)
            def _():
                wait_out(b)

            gb, ob = gbufs[b], obufs[b]

            @plsc.parallel_loop(0, CHUNK, unroll=8)
            def _(r):
                for c in range(D_MODEL // LANES):
                    slc = pl.ds(c * LANES, LANES)
                    ob[r, slc] = gb[r, slc] * SCALE

            start_out(k, b)

            @pl.when(k + NBUF < NCHUNKS)
            def _():
                start_gather(k + NBUF, b)

    for b in range(NBUF):
        wait_out(b)


@jax.jit
def kernel(x, lut):
    idx = x.reshape(B_TOTAL)
    mesh = plsc.VectorSubcoreMesh(core_axis_name="c", subcore_axis_name="s")
    run = pl.kernel(
        _emb_kernel,
        out_type=jax.ShapeDtypeStruct((B_TOTAL, D_MODEL), jnp.float32),
        mesh=mesh,
        scratch_types=[
            pltpu.VMEM((ROWS_PER_WORKER,), jnp.int32),
            pltpu.VMEM((CHUNK, D_MODEL), jnp.float32),
            pltpu.VMEM((CHUNK, D_MODEL), jnp.float32),
            pltpu.VMEM((CHUNK, D_MODEL), jnp.float32),
            pltpu.VMEM((CHUNK, D_MODEL), jnp.float32),
            pltpu.SemaphoreType.DMA,
            pltpu.SemaphoreType.DMA,
            pltpu.SemaphoreType.DMA,
            pltpu.SemaphoreType.DMA,
        ],
        compiler_params=pltpu.CompilerParams(use_tc_tiling_on_sc=False),
    )
    out = run(lut, idx)
    return out.reshape(x.shape[0], x.shape[1], D_MODEL)
